# baseline (speedup 1.0000x reference)
import functools

import jax
import jax.numpy as jnp
from jax import lax
from jax.experimental import pallas as pl
from jax.experimental.pallas import tpu as pltpu
from jax.experimental.pallas import tpu_sc as plsc

SEQ = 8192
DIM = 64
EPS = 1e-5
NC = 2
NS = 16
NW = NC * NS
PER = SEQ // NW
LANES = 16
NGRP = PER // LANES
SUB = 4
NK = DIM // LANES


def _lanesum(v):
    iota = lax.iota(jnp.int32, LANES)
    for sh in (1, 2, 4, 8):
        perm = jnp.bitwise_xor(iota, sh)
        v = v + v.at[perm].get(mode="promise_in_bounds")
    return v


def _rsqrt(v):
    bits = lax.bitcast_convert_type(v, jnp.int32)
    y = lax.bitcast_convert_type(jnp.int32(0x5F3759DF) - (bits >> 1), jnp.float32)
    for _ in range(3):
        y = y * (1.5 - 0.5 * v * y * y)
    return y


def _sc_body(ids_hbm, tblT_hbm, pos_hbm, gamma_hbm, beta_hbm, out_hbm,
             idx_v, gbuf, pos_v, g_v, b_v, sems):
    wid = lax.axis_index("s") * NC + lax.axis_index("c")
    base = wid * PER

    pltpu.sync_copy(ids_hbm.at[pl.ds(base, PER)], idx_v)
    pltpu.sync_copy(pos_hbm.at[pl.ds(base, PER)], pos_v)
    pltpu.sync_copy(gamma_hbm, g_v)
    pltpu.sync_copy(beta_hbm, b_v)

    iota = lax.iota(jnp.int32, LANES)
    dvecs = [(jnp.int32(k * LANES) + iota) for k in range(NK)]
    gvecs = [g_v[pl.ds(k * LANES, LANES)] for k in range(NK)]
    bvecs = [b_v[pl.ds(k * LANES, LANES)] for k in range(NK)]

    def issue(slot, ts):
        for j, t in enumerate(ts):
            off = pl.multiple_of((t >> 7) * 128, 128)
            pltpu.async_copy(tblT_hbm.at[:, pl.ds(off, 128)],
                             gbuf.at[slot, j], sems.at[slot])

    def wait(slot):
        for j in range(SUB):
            pltpu.make_async_copy(tblT_hbm.at[:, pl.ds(0, 128)],
                                  gbuf.at[slot, j], sems.at[slot]).wait()

    def process(slot, ts, jj0):
        for j, t in enumerate(ts):
            jj = jj0 + j
            lvec = jnp.full((LANES,), t & 127, dtype=jnp.int32)
            x = []
            for k in range(NK):
                vals = plsc.load_gather(gbuf.at[slot, j], [dvecs[k], lvec])
                x.append(vals + pos_v[jj, pl.ds(k * LANES, LANES)])
            total = x[0] + x[1] + x[2] + x[3]
            mean = _lanesum(total) * (1.0 / DIM)
            c = [xk - mean for xk in x]
            sq = c[0] * c[0] + c[1] * c[1] + c[2] * c[2] + c[3] * c[3]
            var = _lanesum(sq) * (1.0 / DIM)
            inv = _rsqrt(var + EPS)
            for k in range(NK):
                pos_v[jj, pl.ds(k * LANES, LANES)] = (
                    c[k] * inv * gvecs[k] + bvecs[k])

    def group(g, carry):
        vec = idx_v[pl.ds(pl.multiple_of(g * LANES, 8), LANES)]
        ts = [vec[j] for j in range(LANES)]
        prev = carry
        for s in range(SUB):
            slot = s & 1
            issue(slot, ts[s * SUB:(s + 1) * SUB])
            if s == 0:
                @pl.when(g > 0)
                def _():
                    wait(1)
                    process(1, prev, (g - 1) * LANES + 3 * SUB)
            else:
                wait(1 - slot)
                process(1 - slot, ts[(s - 1) * SUB:s * SUB],
                        g * LANES + (s - 1) * SUB)
        return tuple(ts[3 * SUB:])

    zero = jnp.int32(0)
    last = lax.fori_loop(0, NGRP, group, (zero, zero, zero, zero))
    wait(1)
    process(1, list(last), (NGRP - 1) * LANES + 3 * SUB)

    pltpu.sync_copy(pos_v, out_hbm.at[pl.ds(base, PER)])


@functools.partial(jax.jit, static_argnames=())
def kernel(token_ids, token_table, pos_table, gamma, beta):
    mesh = plsc.VectorSubcoreMesh(core_axis_name="c", subcore_axis_name="s")
    run = functools.partial(
        pl.kernel,
        mesh=mesh,
        compiler_params=pltpu.CompilerParams(needs_layout_passes=False),
        out_type=jax.ShapeDtypeStruct((SEQ, DIM), jnp.float32),
        scratch_types=[
            pltpu.VMEM((PER,), jnp.int32),
            pltpu.VMEM((2, SUB, DIM, 128), jnp.float32),
            pltpu.VMEM((PER, DIM), jnp.float32),
            pltpu.VMEM((DIM,), jnp.float32),
            pltpu.VMEM((DIM,), jnp.float32),
            pltpu.SemaphoreType.DMA((2,)),
        ],
    )(_sc_body)
    return run(token_ids.astype(jnp.int32), token_table.T, pos_table,
               gamma, beta)


# speedup vs baseline: 1.0230x; 1.0230x over previous
import functools

import jax
import jax.numpy as jnp
from jax import lax
from jax.experimental import pallas as pl
from jax.experimental.pallas import tpu as pltpu
from jax.experimental.pallas import tpu_sc as plsc

SEQ = 8192
DIM = 64
EPS = 1e-5
NC = 2
NS = 16
NW = NC * NS
PER = SEQ // NW
LANES = 16
NGRP = PER // LANES
SUB = 4
NK = DIM // LANES


def _lanesum(v):
    iota = lax.iota(jnp.int32, LANES)
    for sh in (1, 2, 4, 8):
        perm = jnp.bitwise_xor(iota, sh)
        v = v + v.at[perm].get(mode="promise_in_bounds")
    return v


def _rsqrt(v):
    bits = lax.bitcast_convert_type(v, jnp.int32)
    y = lax.bitcast_convert_type(jnp.int32(0x5F3759DF) - (bits >> 1), jnp.float32)
    for _ in range(3):
        y = y * (1.5 - 0.5 * v * y * y)
    return y


def _sc_body(ids_hbm, tblT_hbm, posT_hbm, gamma_hbm, beta_hbm, outT_hbm,
             idx_v, gbuf, pv, g_v, b_v, sems):
    wid = lax.axis_index("s") * NC + lax.axis_index("c")
    base = wid * PER

    pltpu.sync_copy(ids_hbm.at[pl.ds(base, PER)], idx_v)
    pltpu.sync_copy(posT_hbm.at[:, pl.ds(base, PER)], pv)
    pltpu.sync_copy(gamma_hbm, g_v)
    pltpu.sync_copy(beta_hbm, b_v)

    iota = lax.iota(jnp.int32, LANES)
    dvecs = [(jnp.int32(k * LANES) + iota) for k in range(NK)]
    gvecs = [g_v[pl.ds(k * LANES, LANES)] for k in range(NK)]
    bvecs = [b_v[pl.ds(k * LANES, LANES)] for k in range(NK)]

    def issue(slot, ts):
        for j, t in enumerate(ts):
            off = pl.multiple_of((t >> 7) * 128, 128)
            pltpu.async_copy(tblT_hbm.at[:, pl.ds(off, 128)],
                             gbuf.at[slot, j], sems.at[slot])

    def wait(slot):
        for j in range(SUB):
            pltpu.make_async_copy(tblT_hbm.at[:, pl.ds(0, 128)],
                                  gbuf.at[slot, j], sems.at[slot]).wait()

    def process(slot, ts, jj0):
        for j, t in enumerate(ts):
            jj = jj0 + j
            lvec = jnp.full((LANES,), t & 127, dtype=jnp.int32)
            jvec = jnp.full((LANES,), jj, dtype=jnp.int32)
            x = []
            for k in range(NK):
                vals = plsc.load_gather(gbuf.at[slot, j], [dvecs[k], lvec])
                x.append(vals + plsc.load_gather(pv, [dvecs[k], jvec]))
            total = x[0] + x[1] + x[2] + x[3]
            mean = _lanesum(total) * (1.0 / DIM)
            c = [xk - mean for xk in x]
            sq = c[0] * c[0] + c[1] * c[1] + c[2] * c[2] + c[3] * c[3]
            var = _lanesum(sq) * (1.0 / DIM)
            inv = _rsqrt(var + EPS)
            for k in range(NK):
                plsc.store_scatter(pv, [dvecs[k], jvec],
                                   c[k] * inv * gvecs[k] + bvecs[k])

    def group(g, carry):
        vec = idx_v[pl.ds(pl.multiple_of(g * LANES, 8), LANES)]
        ts = [vec[j] for j in range(LANES)]
        prev = carry
        for s in range(SUB):
            slot = s & 1
            issue(slot, ts[s * SUB:(s + 1) * SUB])
            if s == 0:
                @pl.when(g > 0)
                def _():
                    wait(1)
                    process(1, prev, (g - 1) * LANES + 3 * SUB)
            else:
                wait(1 - slot)
                process(1 - slot, ts[(s - 1) * SUB:s * SUB],
                        g * LANES + (s - 1) * SUB)
        return tuple(ts[3 * SUB:])

    zero = jnp.int32(0)
    last = lax.fori_loop(0, NGRP, group, (zero, zero, zero, zero))
    wait(1)
    process(1, list(last), (NGRP - 1) * LANES + 3 * SUB)

    pltpu.sync_copy(pv, outT_hbm.at[:, pl.ds(base, PER)])


@functools.partial(jax.jit, static_argnames=())
def kernel(token_ids, token_table, pos_table, gamma, beta):
    mesh = plsc.VectorSubcoreMesh(core_axis_name="c", subcore_axis_name="s")
    run = functools.partial(
        pl.kernel,
        mesh=mesh,
        compiler_params=pltpu.CompilerParams(needs_layout_passes=False),
        out_type=jax.ShapeDtypeStruct((DIM, SEQ), jnp.float32),
        scratch_types=[
            pltpu.VMEM((PER,), jnp.int32),
            pltpu.VMEM((2, SUB, DIM, 128), jnp.float32),
            pltpu.VMEM((DIM, PER), jnp.float32),
            pltpu.VMEM((DIM,), jnp.float32),
            pltpu.VMEM((DIM,), jnp.float32),
            pltpu.SemaphoreType.DMA((2,)),
        ],
    )(_sc_body)
    outT = run(token_ids.astype(jnp.int32), token_table.T, pos_table.T,
               gamma, beta)
    return outT.T


# speedup vs baseline: 1.0649x; 1.0410x over previous
import functools

import jax
import jax.numpy as jnp
from jax import lax
from jax.experimental import pallas as pl
from jax.experimental.pallas import tpu as pltpu
from jax.experimental.pallas import tpu_sc as plsc

SEQ = 8192
DIM = 64
EPS = 1e-5
NC = 2
NS = 16
NW = NC * NS
PER = SEQ // NW
LANES = 16
NGRP = PER // LANES
SUB = 2
RING = 4
NSUB = LANES // SUB
NK = DIM // LANES


def _lanesum(v):
    iota = lax.iota(jnp.int32, LANES)
    for sh in (1, 2, 4, 8):
        perm = jnp.bitwise_xor(iota, sh)
        v = v + v.at[perm].get(mode="promise_in_bounds")
    return v


def _rsqrt(v):
    bits = lax.bitcast_convert_type(v, jnp.int32)
    y = lax.bitcast_convert_type(jnp.int32(0x5F3759DF) - (bits >> 1), jnp.float32)
    for _ in range(3):
        y = y * (1.5 - 0.5 * v * y * y)
    return y


def _sc_body(ids_hbm, tblT_hbm, posT_hbm, gamma_hbm, beta_hbm, outT_hbm,
             idx_v, gbuf, pv, g_v, b_v, sems):
    wid = lax.axis_index("s") * NC + lax.axis_index("c")
    base = wid * PER

    pltpu.sync_copy(ids_hbm.at[pl.ds(base, PER)], idx_v)
    pltpu.sync_copy(posT_hbm.at[:, pl.ds(base, PER)], pv)
    pltpu.sync_copy(gamma_hbm, g_v)
    pltpu.sync_copy(beta_hbm, b_v)

    iota = lax.iota(jnp.int32, LANES)
    dvecs = [(jnp.int32(k * LANES) + iota) for k in range(NK)]
    gvecs = [g_v[pl.ds(k * LANES, LANES)] for k in range(NK)]
    bvecs = [b_v[pl.ds(k * LANES, LANES)] for k in range(NK)]

    def issue(slot, ts):
        for j, t in enumerate(ts):
            off = pl.multiple_of((t >> 7) * 128, 128)
            pltpu.async_copy(tblT_hbm.at[:, pl.ds(off, 128)],
                             gbuf.at[slot, j], sems.at[slot])

    def wait(slot):
        for j in range(SUB):
            pltpu.make_async_copy(tblT_hbm.at[:, pl.ds(0, 128)],
                                  gbuf.at[slot, j], sems.at[slot]).wait()

    def process(slot, ts, jj0):
        for j, t in enumerate(ts):
            jj = jj0 + j
            lvec = jnp.full((LANES,), t & 127, dtype=jnp.int32)
            jvec = jnp.full((LANES,), jj, dtype=jnp.int32)
            x = []
            for k in range(NK):
                vals = plsc.load_gather(gbuf.at[slot, j], [dvecs[k], lvec])
                x.append(vals + plsc.load_gather(pv, [dvecs[k], jvec]))
            total = x[0] + x[1] + x[2] + x[3]
            mean = _lanesum(total) * (1.0 / DIM)
            c = [xk - mean for xk in x]
            sq = c[0] * c[0] + c[1] * c[1] + c[2] * c[2] + c[3] * c[3]
            var = _lanesum(sq) * (1.0 / DIM)
            inv = _rsqrt(var + EPS)
            for k in range(NK):
                plsc.store_scatter(pv, [dvecs[k], jvec],
                                   c[k] * inv * gvecs[k] + bvecs[k])

    def group(g, carry):
        vec = idx_v[pl.ds(pl.multiple_of(g * LANES, 8), LANES)]
        ts = [vec[j] for j in range(LANES)]
        prev = list(carry)
        for s in range(NSUB):
            slot = s & (RING - 1)
            issue(slot, ts[s * SUB:(s + 1) * SUB])
            if s <= 1:
                pslot = NSUB - 2 + s - 4
                @pl.when(g > 0)
                def _(s=s, pslot=pslot):
                    wait(pslot)
                    process(pslot, prev[s * SUB:(s + 1) * SUB],
                            (g - 1) * LANES + (NSUB - 2 + s) * SUB)
            else:
                pslot = (s - 2) & (RING - 1)
                wait(pslot)
                process(pslot, ts[(s - 2) * SUB:(s - 1) * SUB],
                        g * LANES + (s - 2) * SUB)
        return tuple(ts[(NSUB - 2) * SUB:])

    zero = jnp.int32(0)
    last = lax.fori_loop(0, NGRP, group, (zero, zero, zero, zero))
    for s in range(2):
        pslot = NSUB - 2 + s - 4
        wait(pslot)
        process(pslot, list(last)[s * SUB:(s + 1) * SUB],
                (NGRP - 1) * LANES + (NSUB - 2 + s) * SUB)

    pltpu.sync_copy(pv, outT_hbm.at[:, pl.ds(base, PER)])


@functools.partial(jax.jit, static_argnames=())
def kernel(token_ids, token_table, pos_table, gamma, beta):
    mesh = plsc.VectorSubcoreMesh(core_axis_name="c", subcore_axis_name="s")
    run = functools.partial(
        pl.kernel,
        mesh=mesh,
        compiler_params=pltpu.CompilerParams(needs_layout_passes=False),
        out_type=jax.ShapeDtypeStruct((DIM, SEQ), jnp.float32),
        scratch_types=[
            pltpu.VMEM((PER,), jnp.int32),
            pltpu.VMEM((RING, SUB, DIM, 128), jnp.float32),
            pltpu.VMEM((DIM, PER), jnp.float32),
            pltpu.VMEM((DIM,), jnp.float32),
            pltpu.VMEM((DIM,), jnp.float32),
            pltpu.SemaphoreType.DMA((RING,)),
        ],
    )(_sc_body)
    outT = run(token_ids.astype(jnp.int32), token_table.T, pos_table.T,
               gamma, beta)
    return outT.T


# speedup vs baseline: 1.1190x; 1.0508x over previous
import functools

import jax
import jax.numpy as jnp
from jax import lax
from jax.experimental import pallas as pl
from jax.experimental.pallas import tpu as pltpu
from jax.experimental.pallas import tpu_sc as plsc

SEQ = 8192
DIM = 64
EPS = 1e-5
NC = 2
NS = 16
NW = NC * NS
PER = SEQ // NW
LANES = 16
NGRP = PER // LANES
SUB = 2
RING = 4
LAG = 3
NSUB = LANES // SUB
NK = DIM // LANES


def _lanesum(v):
    iota = lax.iota(jnp.int32, LANES)
    for sh in (1, 2, 4, 8):
        perm = jnp.bitwise_xor(iota, sh)
        v = v + v.at[perm].get(mode="promise_in_bounds")
    return v


def _rsqrt(v):
    bits = lax.bitcast_convert_type(v, jnp.int32)
    y = lax.bitcast_convert_type(jnp.int32(0x5F3759DF) - (bits >> 1), jnp.float32)
    for _ in range(3):
        y = y * (1.5 - 0.5 * v * y * y)
    return y


def _sc_body(ids_hbm, tblT_hbm, posT_hbm, gamma_hbm, beta_hbm, outT_hbm,
             idx_v, gbuf, pv, g_v, b_v, sems):
    wid = lax.axis_index("s") * NC + lax.axis_index("c")
    base = wid * PER

    pltpu.sync_copy(ids_hbm.at[pl.ds(base, PER)], idx_v)
    pltpu.sync_copy(posT_hbm.at[:, pl.ds(base, PER)], pv)
    pltpu.sync_copy(gamma_hbm, g_v)
    pltpu.sync_copy(beta_hbm, b_v)

    iota = lax.iota(jnp.int32, LANES)
    dvecs = [(jnp.int32(k * LANES) + iota) for k in range(NK)]
    gvecs = [g_v[pl.ds(k * LANES, LANES)] for k in range(NK)]
    bvecs = [b_v[pl.ds(k * LANES, LANES)] for k in range(NK)]

    def issue(slot, ts):
        for j, t in enumerate(ts):
            off = pl.multiple_of((t >> 7) * 128, 128)
            pltpu.async_copy(tblT_hbm.at[:, pl.ds(off, 128)],
                             gbuf.at[slot, j], sems.at[slot])

    def wait(slot):
        for j in range(SUB):
            pltpu.make_async_copy(tblT_hbm.at[:, pl.ds(0, 128)],
                                  gbuf.at[slot, j], sems.at[slot]).wait()

    def process(slot, ts, jj0):
        for j, t in enumerate(ts):
            jj = jj0 + j
            lvec = jnp.full((LANES,), t & 127, dtype=jnp.int32)
            jvec = jnp.full((LANES,), jj, dtype=jnp.int32)
            x = []
            for k in range(NK):
                vals = plsc.load_gather(gbuf.at[slot, j], [dvecs[k], lvec])
                x.append(vals + plsc.load_gather(pv, [dvecs[k], jvec]))
            total = x[0] + x[1] + x[2] + x[3]
            mean = _lanesum(total) * (1.0 / DIM)
            c = [xk - mean for xk in x]
            sq = c[0] * c[0] + c[1] * c[1] + c[2] * c[2] + c[3] * c[3]
            var = _lanesum(sq) * (1.0 / DIM)
            inv = _rsqrt(var + EPS)
            for k in range(NK):
                plsc.store_scatter(pv, [dvecs[k], jvec],
                                   c[k] * inv * gvecs[k] + bvecs[k])

    def group(g, carry):
        vec = idx_v[pl.ds(pl.multiple_of(g * LANES, 8), LANES)]
        ts = [vec[j] for j in range(LANES)]
        prev = list(carry)
        for s in range(NSUB):
            slot = s & (RING - 1)
            issue(slot, ts[s * SUB:(s + 1) * SUB])
            if s <= LAG - 1:
                q = NSUB - LAG + s
                pslot = q & (RING - 1)
                @pl.when(g > 0)
                def _(q=q, pslot=pslot, s=s):
                    wait(pslot)
                    process(pslot, prev[s * SUB:(s + 1) * SUB],
                            (g - 1) * LANES + q * SUB)
            else:
                q = s - LAG
                pslot = q & (RING - 1)
                wait(pslot)
                process(pslot, ts[q * SUB:(q + 1) * SUB],
                        g * LANES + q * SUB)
        return tuple(ts[(NSUB - LAG) * SUB:])

    zero = jnp.int32(0)
    last = lax.fori_loop(0, NGRP, group, (zero,) * (LAG * SUB))
    for s in range(LAG):
        q = NSUB - LAG + s
        pslot = q & (RING - 1)
        wait(pslot)
        process(pslot, list(last)[s * SUB:(s + 1) * SUB],
                (NGRP - 1) * LANES + q * SUB)

    pltpu.sync_copy(pv, outT_hbm.at[:, pl.ds(base, PER)])


@functools.partial(jax.jit, static_argnames=())
def kernel(token_ids, token_table, pos_table, gamma, beta):
    mesh = plsc.VectorSubcoreMesh(core_axis_name="c", subcore_axis_name="s")
    run = functools.partial(
        pl.kernel,
        mesh=mesh,
        compiler_params=pltpu.CompilerParams(needs_layout_passes=False),
        out_type=jax.ShapeDtypeStruct((DIM, SEQ), jnp.float32),
        scratch_types=[
            pltpu.VMEM((PER,), jnp.int32),
            pltpu.VMEM((RING, SUB, DIM, 128), jnp.float32),
            pltpu.VMEM((DIM, PER), jnp.float32),
            pltpu.VMEM((DIM,), jnp.float32),
            pltpu.VMEM((DIM,), jnp.float32),
            pltpu.SemaphoreType.DMA((RING,)),
        ],
    )(_sc_body)
    outT = run(token_ids.astype(jnp.int32), token_table.T, pos_table.T,
               gamma, beta)
    return outT.T
